# baseline (speedup 1.0000x reference)
import functools
import jax
import jax.numpy as jnp
from jax import lax
from jax.experimental import pallas as pl
from jax.experimental.pallas import tpu as pltpu
from jax.experimental.pallas import tpu_sc as plsc

_BLK_B = 256

_NN = 50000
_E_REAL = 800000
_E_PAD = 800256
_NDEN = 50048
_KB = 128
_B = 1024
_BP = 1032
_NPOOL = 51200


def _lrelu(x, s=0.01):
    return jnp.where(x > 0, x, s * x)


def _sigmoid(x):
    return 1.0 / (1.0 + jnp.exp(-x))


_BN_SCALE = float((1.0 + 1e-5) ** -0.5)


def _tail_body(x1g, x2g, f1, f2, cv, *refs):
    (out_ref,) = refs[-1:]
    params = refs[:-1]
    it = iter(params)

    feats = [x1g[...], x2g[...], f1[...], f2[...], cv[...]]
    outs = list(feats)
    for s in range(5):
        x = outs[s]
        for li in range(2):
            wg, bg, wn, bn_, wl, bl = (next(it)[...] for _ in range(6))
            g = _sigmoid(jnp.dot(x, wg, preferred_element_type=jnp.float32) + bg)
            nl = _lrelu(jnp.dot(x, wn, preferred_element_type=jnp.float32) + bn_)
            ln = jnp.dot(x, wl, preferred_element_type=jnp.float32) + bl
            x = g * nl + (1.0 - g) * ln + x
        outs[s] = x
    wgp, bgp, ggp, bbgp = (next(it)[...] for _ in range(4))
    gc = jnp.concatenate(outs, axis=1)
    gpo = jnp.dot(gc, wgp, preferred_element_type=jnp.float32) + bgp
    gpo = jnp.maximum(gpo * _BN_SCALE * ggp + bbgp, 0.0)
    w1, b1, w2, b2 = (next(it)[...] for _ in range(4))
    m = jnp.stack([f.mean(axis=1) for f in feats], axis=1)
    w = _sigmoid(jnp.dot(jnp.maximum(jnp.dot(m, w1, preferred_element_type=jnp.float32) + b1, 0.0),
                         w2, preferred_element_type=jnp.float32) + b2)
    se = sum(feats[s] * w[:, s:s + 1] for s in range(5))
    ws1, bs1, ws2, bs2, ws3, bs3 = (next(it)[...] for _ in range(6))
    dual = jnp.concatenate([gpo, se], axis=1)
    h = _lrelu(jnp.dot(dual, ws1, preferred_element_type=jnp.float32) + bs1)
    h = _lrelu(jnp.dot(h, ws2, preferred_element_type=jnp.float32) + bs2)
    out_ref[...] = jnp.dot(h, ws3, preferred_element_type=jnp.float32) + bs3


def _tail(x1g, x2g, f1, f2, cv, params):
    b = x1g.shape[0]
    plist = []
    for s in ['d1g', 'd2g', 'd1f', 'd2f', 'cell']:
        for li in range(2):
            lp = params['mpgr'][s][li]
            plist += [lp['gate']['w'], lp['gate']['b'], lp['nl']['w'], lp['nl']['b'],
                      lp['lin']['w'], lp['lin']['b']]
    gp = params['gp']
    plist += [gp['l']['w'], gp['l']['b'], gp['bn']['g'], gp['bn']['b']]
    se = params['se']
    plist += [se['l1']['w'], se['l1']['b'], se['l2']['w'], se['l2']['b']]
    syn = params['syn']
    plist += [syn['l1']['w'], syn['l1']['b'], syn['l2']['w'], syn['l2']['b'],
              syn['l3']['w'], syn['l3']['b']]

    grid = (b // _BLK_B,)
    feat_spec = pl.BlockSpec((_BLK_B, 128), lambda i: (i, 0))
    pspecs = [pl.BlockSpec(p.shape, lambda i, _r=len(p.shape): (0,) * _r) for p in plist]
    return pl.pallas_call(
        _tail_body,
        grid=grid,
        in_specs=[feat_spec] * 5 + pspecs,
        out_specs=pl.BlockSpec((_BLK_B, 2), lambda i: (i, 0)),
        out_shape=jax.ShapeDtypeStruct((b, 2), jnp.float32),
    )(x1g, x2g, f1, f2, cv, *plist)


@functools.lru_cache(maxsize=None)
def _make_edge_aggr(dim):
    S = dim // 32
    split_edges = (S == 1)
    n_out = 2 if split_edges else S
    n_den = 2 if split_edges else 1
    n_workers = 32 if split_edges else 16
    P = _E_PAD // n_workers
    nblk = P // _KB
    tail = P - nblk * _KB
    spc = 1 if split_edges else S // 2
    mesh = plsc.VectorSubcoreMesh(core_axis_name="c", subcore_axis_name="s")
    out_type = tuple([jax.ShapeDtypeStruct((n_out, _NDEN, 32), jnp.float32)] +
                     [jax.ShapeDtypeStruct((_NDEN,), jnp.float32)] * n_den)
    scratch = [
        pltpu.VMEM_SHARED((_NDEN, 32), jnp.float32),
        pltpu.VMEM_SHARED((_NDEN,), jnp.float32),
        pltpu.VMEM((256, 32), jnp.float32),
        pltpu.VMEM((256,), jnp.float32),
        pltpu.VMEM((_KB,), jnp.int32),
        pltpu.VMEM((_KB,), jnp.int32),
        pltpu.VMEM((_KB,), jnp.float32),
        pltpu.VMEM((_KB,), jnp.float32),
        pltpu.VMEM((_KB,), jnp.float32),
        pltpu.VMEM((_KB, 32), jnp.float32),
        pltpu.VMEM((tail,), jnp.int32),
        pltpu.VMEM((tail,), jnp.int32),
        pltpu.VMEM((tail,), jnp.float32),
        pltpu.VMEM((tail,), jnp.float32),
        pltpu.VMEM((tail,), jnp.float32),
        pltpu.VMEM((tail, 32), jnp.float32),
        pltpu.VMEM((16,), jnp.float32),
        pltpu.SemaphoreType.DMA,
    ]

    @functools.partial(pl.kernel, out_type=out_type, mesh=mesh,
                       scratch_types=scratch,
                       compiler_params=pltpu.CompilerParams(
                           use_tc_tiling_on_sc=False))
    def body(src_h, dst_h, as_h, ad_h, m_h, *rest):
        h_refs = rest[:S]
        acc_o = rest[S]
        den_os = rest[S + 1:S + 1 + n_den]
        (acc_sp, den_sp, zrow, zflat, src_b, dst_b, as_b, ad_b, e_b, rows_b,
         src_t, dst_t, as_t, ad_t, e_t, rows_t, m_v, sem) = rest[S + 1 + n_den:]
        c = lax.axis_index("c")
        s = lax.axis_index("s")

        pltpu.sync_copy(m_h, m_v)

        def zfill(i, _):
            zrow[i, pl.ds(0, 16)] = jnp.zeros((16,), jnp.float32)
            zrow[i, pl.ds(16, 16)] = jnp.zeros((16,), jnp.float32)
            return 0
        lax.fori_loop(0, 256, zfill, 0)

        def zfill1(i, _):
            zflat[pl.ds(i * 16, 16)] = jnp.zeros((16,), jnp.float32)
            return 0
        lax.fori_loop(0, 16, zfill1, 0)

        chunks = [(i * 256, 256) for i in range(12)] + [(3072, 56)]

        def _stripe(off, sz):
            return pl.multiple_of(s * 3128 + off, 8), sz

        def zero_acc():
            for off, sz in chunks:
                o, _ = _stripe(off, sz)
                pltpu.sync_copy(zrow.at[pl.ds(0, sz)],
                                acc_sp.at[pl.ds(o, sz)])

        def zero_den():
            for off, sz in chunks:
                o, _ = _stripe(off, sz)
                pltpu.sync_copy(zflat.at[pl.ds(0, sz)],
                                den_sp.at[pl.ds(o, sz)])

        def copy_acc(out_idx):
            for off, sz in chunks:
                o, _ = _stripe(off, sz)
                pltpu.sync_copy(acc_sp.at[pl.ds(o, sz)],
                                acc_o.at[out_idx, pl.ds(o, sz)])

        def copy_den(den_idx):
            for off, sz in chunks:
                o, _ = _stripe(off, sz)
                pltpu.sync_copy(den_sp.at[pl.ds(o, sz)],
                                den_os[den_idx].at[pl.ds(o, sz)])

        def process(base, kk, sb, db, ab, bb, eb, rb, hsl_ref, with_den):
            base = pl.multiple_of(base, 8)
            pltpu.sync_copy(src_h.at[pl.ds(base, kk)], sb)
            pltpu.sync_copy(dst_h.at[pl.ds(base, kk)], db)
            pltpu.async_copy(as_h.at[sb], ab, sem).wait()
            pltpu.async_copy(ad_h.at[db], bb, sem).wait()
            m = m_v[pl.ds(0, 16)][0]
            for j in range(kk // 16):
                z = ab[pl.ds(j * 16, 16)] + bb[pl.ds(j * 16, 16)]
                a = jnp.maximum(z, 0.0) + 0.2 * jnp.minimum(z, 0.0)
                e = jnp.exp(a - m)
                gidx = (base + j * 16) + lax.iota(jnp.int32, 16)
                e = jnp.where(gidx < _E_REAL, e, jnp.zeros((16,), jnp.float32))
                eb[pl.ds(j * 16, 16)] = e
            if with_den:
                pltpu.sync_copy(eb, den_sp.at[db], add=True)
            pltpu.async_copy(hsl_ref.at[sb], rb, sem).wait()

            def scale(j, _):
                ev = eb[pl.ds(j * 16, 16)]
                for k2 in range(16):
                    r = j * 16 + k2
                    ek = ev[k2]
                    rb[r, pl.ds(0, 16)] = rb[r, pl.ds(0, 16)] * ek
                    rb[r, pl.ds(16, 16)] = rb[r, pl.ds(16, 16)] * ek
                return 0
            lax.fori_loop(0, kk // 16, scale, 0)
            pltpu.sync_copy(rb, acc_sp.at[db], add=True)

        def run_pass(hsl_ref, out_idx, with_den, den_idx, base0):
            zero_acc()
            if with_den:
                zero_den()
            plsc.subcore_barrier()

            def blk(i, _):
                process(base0 + i * _KB, _KB, src_b, dst_b, as_b, ad_b, e_b,
                        rows_b, hsl_ref, with_den)
                return 0
            lax.fori_loop(0, nblk, blk, 0)
            if tail:
                process(base0 + nblk * _KB, tail, src_t, dst_t, as_t, ad_t,
                        e_t, rows_t, hsl_ref, with_den)
            plsc.subcore_barrier()
            copy_acc(out_idx)
            if with_den:
                copy_den(den_idx)
            plsc.subcore_barrier()

        for cc in range(2):
            @pl.when(c == cc)
            def _(cc=cc):
                if split_edges:
                    run_pass(h_refs[0], out_idx=cc, with_den=True, den_idx=cc,
                             base0=(s * 2 + cc) * P)
                else:
                    for si in range(spc):
                        k_idx = cc * spc + si
                        run_pass(h_refs[k_idx], out_idx=k_idx,
                                 with_den=(cc == 0 and si == 0), den_idx=0,
                                 base0=s * P)

    return body


@functools.lru_cache(maxsize=None)
def _make_pool():
    rows_per_w = _NPOOL // 32
    nblk = rows_per_w // _KB
    tail = rows_per_w - nblk * _KB
    mesh = plsc.VectorSubcoreMesh(core_axis_name="c", subcore_axis_name="s")
    out_type = (jax.ShapeDtypeStruct((2, _BP, 128), jnp.float32),
                jax.ShapeDtypeStruct((_BP,), jnp.float32),
                jax.ShapeDtypeStruct((_BP,), jnp.float32))
    scratch = [
        pltpu.VMEM_SHARED((_BP, 128), jnp.float32),
        pltpu.VMEM_SHARED((_BP,), jnp.float32),
        pltpu.VMEM((64, 128), jnp.float32),
        pltpu.VMEM((256,), jnp.float32),
        pltpu.VMEM((_KB, 128), jnp.float32),
        pltpu.VMEM((_KB,), jnp.int32),
        pltpu.VMEM((_KB,), jnp.float32),
        pltpu.VMEM((tail, 128), jnp.float32),
        pltpu.VMEM((tail,), jnp.int32),
        pltpu.VMEM((tail,), jnp.float32),
        pltpu.SemaphoreType.DMA,
    ]

    @functools.partial(pl.kernel, out_type=out_type, mesh=mesh,
                       scratch_types=scratch)
    def body(x_h, b_h, acc_o, cnt_o0, cnt_o1, acc_sp, cnt_sp, zp, zf, rows,
             bidx, ones_v, rows_t, bidx_t, ones_t, sem):
        cnt_os = (cnt_o0, cnt_o1)
        c = lax.axis_index("c")
        s = lax.axis_index("s")

        def zfill(i, _):
            for j in range(8):
                zp[i, pl.ds(j * 16, 16)] = jnp.zeros((16,), jnp.float32)
            return 0
        lax.fori_loop(0, 64, zfill, 0)

        def zfill1(i, _):
            zf[pl.ds(i * 16, 16)] = jnp.zeros((16,), jnp.float32)
            return 0
        lax.fori_loop(0, 16, zfill1, 0)

        for j in range(_KB // 16):
            ones_v[pl.ds(j * 16, 16)] = jnp.ones((16,), jnp.float32)
        for j in range(tail // 16):
            ones_t[pl.ds(j * 16, 16)] = jnp.ones((16,), jnp.float32)

        @pl.when(s == 0)
        def _():
            for i in range(16):
                pltpu.sync_copy(zp, acc_sp.at[pl.ds(i * 64, 64)])
            pltpu.sync_copy(zp.at[pl.ds(0, 8)], acc_sp.at[pl.ds(1024, 8)])
            for i in range(4):
                pltpu.sync_copy(zf, cnt_sp.at[pl.ds(i * 256, 256)])
            pltpu.sync_copy(zf.at[pl.ds(0, 8)], cnt_sp.at[pl.ds(1024, 8)])
        plsc.subcore_barrier()

        def do_block(rb, kk, rbuf, ibuf, obuf):
            rb = pl.multiple_of(rb, 8)
            pltpu.sync_copy(x_h.at[pl.ds(rb, kk)], rbuf)
            pltpu.sync_copy(b_h.at[pl.ds(rb, kk)], ibuf)
            pltpu.sync_copy(rbuf, acc_sp.at[ibuf], add=True)
            pltpu.sync_copy(obuf, cnt_sp.at[ibuf], add=True)

        for cc in range(2):
            @pl.when(c == cc)
            def _(cc=cc):
                base0 = (s * 2 + cc) * rows_per_w

                def blk(i, _):
                    do_block(base0 + i * _KB, _KB, rows, bidx, ones_v)
                    return 0
                lax.fori_loop(0, nblk, blk, 0)
                do_block(base0 + nblk * _KB, tail, rows_t, bidx_t, ones_t)
                plsc.subcore_barrier()

                @pl.when(s == 0)
                def _():
                    pltpu.sync_copy(acc_sp, acc_o.at[cc])
                    pltpu.sync_copy(cnt_sp, cnt_os[cc])

    return body


_BLK_N = 400


def _dot(a, w):
    return jnp.dot(a, w, preferred_element_type=jnp.float32)


def _gat_pre_body(x_ref, w_ref, as_ref, ad_ref, h_ref, aux_ref, mx_ref):
    h = _dot(x_ref[...], w_ref[...])
    h_ref[...] = h
    asum = _dot(h, as_ref[...])
    adsum = _dot(h, ad_ref[...])
    aux = jnp.concatenate([asum, adsum, jnp.zeros_like(asum),
                           jnp.zeros_like(asum)], axis=1)
    aux_ref[...] = aux
    mx_ref[...] = jnp.concatenate([jnp.max(asum).reshape(1, 1, 1),
                                   jnp.max(adsum).reshape(1, 1, 1)], axis=2)


def _gat_pre(x, p):
    n, d_in = x.shape
    dim = p['w'].shape[1]
    grid = (n // _BLK_N,)
    h, aux, mx = pl.pallas_call(
        _gat_pre_body,
        grid=grid,
        in_specs=[pl.BlockSpec((_BLK_N, d_in), lambda i: (i, 0)),
                  pl.BlockSpec((d_in, dim), lambda i: (0, 0)),
                  pl.BlockSpec((dim, 1), lambda i: (0, 0)),
                  pl.BlockSpec((dim, 1), lambda i: (0, 0))],
        out_specs=[pl.BlockSpec((_BLK_N, dim), lambda i: (i, 0)),
                   pl.BlockSpec((_BLK_N, 4), lambda i: (i, 0)),
                   pl.BlockSpec((1, 1, 2), lambda i: (i, 0, 0))],
        out_shape=[jax.ShapeDtypeStruct((n, dim), jnp.float32),
                   jax.ShapeDtypeStruct((n, 4), jnp.float32),
                   jax.ShapeDtypeStruct((grid[0], 1, 2), jnp.float32)],
    )(x, p['w'], p['as'][:, None], p['ad'][:, None])
    return h, aux, mx


def _gat_post_body(acc_ref, den_ref, h_ref, aux_ref, m_ref, b_ref, o_ref):
    m = m_ref[0, 0]
    z = aux_ref[:, 0:1] + aux_ref[:, 1:2]
    e_self = jnp.exp(jnp.where(z > 0, z, 0.2 * z) - m)
    num = acc_ref[...] + e_self * h_ref[...]
    den = den_ref[...] + e_self + 1e-16
    out = num / den + b_ref[...]
    o_ref[...] = jnp.where(out > 0, out, 0.01 * out)


def _gat_post(acc, den, h, aux, m, b):
    n, dim = h.shape
    grid = (n // _BLK_N,)
    return pl.pallas_call(
        _gat_post_body,
        grid=grid,
        in_specs=[pl.BlockSpec((_BLK_N, dim), lambda i: (i, 0)),
                  pl.BlockSpec((_BLK_N, 1), lambda i: (i, 0)),
                  pl.BlockSpec((_BLK_N, dim), lambda i: (i, 0)),
                  pl.BlockSpec((_BLK_N, 4), lambda i: (i, 0)),
                  pl.BlockSpec((1, 1), lambda i: (0, 0)),
                  pl.BlockSpec((1, dim), lambda i: (0, 0))],
        out_specs=pl.BlockSpec((_BLK_N, dim), lambda i: (i, 0)),
        out_shape=jax.ShapeDtypeStruct((n, dim), jnp.float32),
    )(acc, den[:, None], h, aux, m[None, None], b[None, :])


def _fem_head_body(ps_ref, cnt_ref, w1, b1, g1, bb1, w2, b2, o_ref):
    pooled = ps_ref[...] / jnp.maximum(cnt_ref[...], 1.0)
    h = _dot(pooled, w1[...]) + b1[...]
    h = h * _BN_SCALE * g1[...] + bb1[...]
    h = jnp.where(h > 0, h, 0.01 * h)
    o_ref[...] = _dot(h, w2[...]) + b2[...]


def _fem_head(psum, cnt, p):
    grid = (_B // _BLK_B,)
    vec = lambda a: a[None, :]
    return pl.pallas_call(
        _fem_head_body,
        grid=grid,
        in_specs=[pl.BlockSpec((_BLK_B, 128), lambda i: (i, 0)),
                  pl.BlockSpec((_BLK_B, 1), lambda i: (i, 0))] +
                 [pl.BlockSpec(s, lambda i: (0, 0)) for s in
                  [(128, 128), (1, 128), (1, 128), (1, 128), (128, 128), (1, 128)]],
        out_specs=pl.BlockSpec((_BLK_B, 128), lambda i: (i, 0)),
        out_shape=jax.ShapeDtypeStruct((_B, 128), jnp.float32),
    )(psum, cnt[:, None], p['fc1']['w'], vec(p['fc1']['b']),
      vec(p['fc1_bn']['g']), vec(p['fc1_bn']['b']),
      p['fc2']['w'], vec(p['fc2']['b']))


def _fp_body(x_ref, w1, b1, g1, bb1, w2, b2, g2, bb2, o_ref):
    h = _dot(x_ref[...], w1[...]) + b1[...]
    h = jnp.maximum(h * _BN_SCALE * g1[...] + bb1[...], 0.0)
    h = _dot(h, w2[...]) + b2[...]
    o_ref[...] = jnp.maximum(h * _BN_SCALE * g2[...] + bb2[...], 0.0)


def _fp_enc_tc(x, p):
    grid = (_B // _BLK_B,)
    vec = lambda a: a[None, :]
    return pl.pallas_call(
        _fp_body,
        grid=grid,
        in_specs=[pl.BlockSpec((_BLK_B, 2048), lambda i: (i, 0))] +
                 [pl.BlockSpec(s, lambda i: (0, 0)) for s in
                  [(2048, 1024), (1, 1024), (1, 1024), (1, 1024),
                   (1024, 128), (1, 128), (1, 128), (1, 128)]],
        out_specs=pl.BlockSpec((_BLK_B, 128), lambda i: (i, 0)),
        out_shape=jax.ShapeDtypeStruct((_B, 128), jnp.float32),
    )(x, p['l1']['w'], vec(p['l1']['b']), vec(p['bn1']['g']), vec(p['bn1']['b']),
      p['l2']['w'], vec(p['l2']['b']), vec(p['bn2']['g']), vec(p['bn2']['b']))


def _cell_body(x_ref, w1, b1, g1, bb1, w2, b2, g2, bb2, w3, b3, o_ref):
    x = x_ref[...]
    nrm = jnp.sqrt(jnp.sum(x * x, axis=1, keepdims=True))
    x = x / jnp.maximum(nrm, 1e-12)
    h = _dot(x, w1[...]) + b1[...]
    h = h * _BN_SCALE * g1[...] + bb1[...]
    h = jnp.where(h > 0, h, 0.01 * h)
    h = _dot(h, w2[...]) + b2[...]
    h = h * _BN_SCALE * g2[...] + bb2[...]
    h = jnp.where(h > 0, h, 0.01 * h)
    o_ref[...] = _dot(h, w3[...]) + b3[...]


def _cell_enc_tc(x, p):
    grid = (_B // _BLK_B,)
    vec = lambda a: a[None, :]
    return pl.pallas_call(
        _cell_body,
        grid=grid,
        in_specs=[pl.BlockSpec((_BLK_B, 954), lambda i: (i, 0))] +
                 [pl.BlockSpec(s, lambda i: (0, 0)) for s in
                  [(954, 256), (1, 256), (1, 256), (1, 256),
                   (256, 128), (1, 128), (1, 128), (1, 128),
                   (128, 128), (1, 128)]],
        out_specs=pl.BlockSpec((_BLK_B, 128), lambda i: (i, 0)),
        out_shape=jax.ShapeDtypeStruct((_B, 128), jnp.float32),
    )(x, p['l1']['w'], vec(p['l1']['b']), vec(p['bn1']['g']), vec(p['bn1']['b']),
      p['l2']['w'], vec(p['l2']['b']), vec(p['bn2']['g']), vec(p['bn2']['b']),
      p['l3']['w'], vec(p['l3']['b']))


def _gat_layer_sc(x, src_p, dst_p, p):
    h, aux, mx = _gat_pre(x, p)
    dim = h.shape[1]
    S = dim // 32
    asum = aux[:, 0]
    adsum = aux[:, 1]
    mb = jnp.max(mx[:, 0, 0]) + jnp.max(mx[:, 0, 1])
    m = jnp.where(mb > 0, mb, 0.2 * mb)
    marr = jnp.full((16,), m, jnp.float32)
    hsl = [h[:, 32 * i:32 * (i + 1)] for i in range(S)]
    res = _make_edge_aggr(dim)(src_p, dst_p, asum, adsum, marr, *hsl)
    acc_p, den_p = res[0], res[1:]
    if S == 1:
        acc = acc_p[0] + acc_p[1]
        den = den_p[0] + den_p[1]
    else:
        acc = jnp.concatenate([acc_p[i] for i in range(S)], axis=1)
        den = den_p[0]
    return _gat_post(acc[:_NN], den[:_NN], h, aux, m, p['b'])


def _fem(x, ei, batch, p, b):
    src_p = jnp.pad(ei[0], (0, _E_PAD - _E_REAL))
    dst_p = jnp.pad(ei[1], (0, _E_PAD - _E_REAL))
    for gp in p['gat']:
        x = _gat_layer_sc(x, src_p, dst_p, gp)
    xp = jnp.pad(x, ((0, _NPOOL - _NN), (0, 0)))
    bp = jnp.pad(batch, (0, _NPOOL - _NN), constant_values=_B)
    acc_p, cnt0, cnt1 = _make_pool()(xp, bp)
    pooled_sum = (acc_p[0] + acc_p[1])[:_B]
    cnt = (cnt0 + cnt1)[:_B]
    return _fem_head(pooled_sum, cnt, p)


def kernel(x1, edge_index1, batch1, fp1, x2, edge_index2, batch2, fp2, cell, params):
    b = fp1.shape[0]
    x1g = _fem(x1, edge_index1, batch1, params['fem1'], b)
    x2g = _fem(x2, edge_index2, batch2, params['fem2'], b)
    f1 = _fp_enc_tc(fp1, params['fp'])
    f2 = _fp_enc_tc(fp2, params['fp'])
    cv = _cell_enc_tc(cell, params['cell'])
    return _tail(x1g, x2g, f1, f2, cv, params)


# speedup vs baseline: 2.0769x; 2.0769x over previous
import functools
import jax
import jax.numpy as jnp
from jax import lax
from jax.experimental import pallas as pl
from jax.experimental.pallas import tpu as pltpu
from jax.experimental.pallas import tpu_sc as plsc

_BLK_B = 256

_NN = 50000
_E_REAL = 800000
_E_PAD = 800768
_NDEN = 50048
_KB = 128
_B = 1024
_BP = 1032
_NPOOL = 51200


def _lrelu(x, s=0.01):
    return jnp.where(x > 0, x, s * x)


def _sigmoid(x):
    return 1.0 / (1.0 + jnp.exp(-x))


_BN_SCALE = float((1.0 + 1e-5) ** -0.5)


def _tail_body(x1g, x2g, f1, f2, cv, *refs):
    (out_ref,) = refs[-1:]
    params = refs[:-1]
    it = iter(params)

    feats = [x1g[...], x2g[...], f1[...], f2[...], cv[...]]
    outs = list(feats)
    for s in range(5):
        x = outs[s]
        for li in range(2):
            wg, bg, wn, bn_, wl, bl = (next(it)[...] for _ in range(6))
            g = _sigmoid(jnp.dot(x, wg, preferred_element_type=jnp.float32) + bg)
            nl = _lrelu(jnp.dot(x, wn, preferred_element_type=jnp.float32) + bn_)
            ln = jnp.dot(x, wl, preferred_element_type=jnp.float32) + bl
            x = g * nl + (1.0 - g) * ln + x
        outs[s] = x
    wgp, bgp, ggp, bbgp = (next(it)[...] for _ in range(4))
    gc = jnp.concatenate(outs, axis=1)
    gpo = jnp.dot(gc, wgp, preferred_element_type=jnp.float32) + bgp
    gpo = jnp.maximum(gpo * _BN_SCALE * ggp + bbgp, 0.0)
    w1, b1, w2, b2 = (next(it)[...] for _ in range(4))
    m = jnp.stack([f.mean(axis=1) for f in feats], axis=1)
    w = _sigmoid(jnp.dot(jnp.maximum(jnp.dot(m, w1, preferred_element_type=jnp.float32) + b1, 0.0),
                         w2, preferred_element_type=jnp.float32) + b2)
    se = sum(feats[s] * w[:, s:s + 1] for s in range(5))
    ws1, bs1, ws2, bs2, ws3, bs3 = (next(it)[...] for _ in range(6))
    dual = jnp.concatenate([gpo, se], axis=1)
    h = _lrelu(jnp.dot(dual, ws1, preferred_element_type=jnp.float32) + bs1)
    h = _lrelu(jnp.dot(h, ws2, preferred_element_type=jnp.float32) + bs2)
    out_ref[...] = jnp.dot(h, ws3, preferred_element_type=jnp.float32) + bs3


def _tail(x1g, x2g, f1, f2, cv, params):
    b = x1g.shape[0]
    plist = []
    for s in ['d1g', 'd2g', 'd1f', 'd2f', 'cell']:
        for li in range(2):
            lp = params['mpgr'][s][li]
            plist += [lp['gate']['w'], lp['gate']['b'], lp['nl']['w'], lp['nl']['b'],
                      lp['lin']['w'], lp['lin']['b']]
    gp = params['gp']
    plist += [gp['l']['w'], gp['l']['b'], gp['bn']['g'], gp['bn']['b']]
    se = params['se']
    plist += [se['l1']['w'], se['l1']['b'], se['l2']['w'], se['l2']['b']]
    syn = params['syn']
    plist += [syn['l1']['w'], syn['l1']['b'], syn['l2']['w'], syn['l2']['b'],
              syn['l3']['w'], syn['l3']['b']]

    grid = (b // _BLK_B,)
    feat_spec = pl.BlockSpec((_BLK_B, 128), lambda i: (i, 0))
    pspecs = [pl.BlockSpec(p.shape, lambda i, _r=len(p.shape): (0,) * _r) for p in plist]
    return pl.pallas_call(
        _tail_body,
        grid=grid,
        in_specs=[feat_spec] * 5 + pspecs,
        out_specs=pl.BlockSpec((_BLK_B, 2), lambda i: (i, 0)),
        out_shape=jax.ShapeDtypeStruct((b, 2), jnp.float32),
    )(x1g, x2g, f1, f2, cv, *plist)


@functools.lru_cache(maxsize=None)
def _make_edge_aggr(dim):
    S = dim // 32
    split_edges = (S == 1)
    n_out = 2 if split_edges else S
    n_den = 2 if split_edges else 1
    n_workers = 32 if split_edges else 16
    P = _E_PAD // n_workers
    nblk = P // _KB
    tail = P - nblk * _KB
    spc = 1 if split_edges else S // 2
    mesh = plsc.VectorSubcoreMesh(core_axis_name="c", subcore_axis_name="s")
    out_type = tuple([jax.ShapeDtypeStruct((n_out, _NDEN, 32), jnp.float32)] +
                     [jax.ShapeDtypeStruct((_NDEN,), jnp.float32)] * n_den)
    blkbufs = [pltpu.VMEM((_KB,), jnp.int32),
               pltpu.VMEM((_KB,), jnp.int32),
               pltpu.VMEM((_KB,), jnp.float32),
               pltpu.VMEM((_KB,), jnp.float32),
               pltpu.VMEM((_KB, 32), jnp.float32)]
    tailbufs = [pltpu.VMEM((tail,), jnp.int32),
                pltpu.VMEM((tail,), jnp.int32),
                pltpu.VMEM((tail,), jnp.float32),
                pltpu.VMEM((tail,), jnp.float32),
                pltpu.VMEM((tail,), jnp.float32),
                pltpu.VMEM((tail, 32), jnp.float32)] if tail else []
    scratch = ([
        pltpu.VMEM_SHARED((_NDEN, 32), jnp.float32),
        pltpu.VMEM_SHARED((_NDEN,), jnp.float32),
        pltpu.VMEM((128, 32), jnp.float32),
        pltpu.VMEM((128,), jnp.float32),
        pltpu.VMEM((_KB,), jnp.float32),
        pltpu.VMEM((16,), jnp.float32)]
        + blkbufs + blkbufs + tailbufs
        + [pltpu.SemaphoreType.DMA, pltpu.SemaphoreType.DMA])

    @functools.partial(pl.kernel, out_type=out_type, mesh=mesh,
                       scratch_types=scratch,
                       compiler_params=pltpu.CompilerParams(
                           use_tc_tiling_on_sc=False))
    def body(src_h, dst_h, as_h, ad_h, m_h, *rest):
        h_refs = rest[:S]
        acc_o = rest[S]
        den_os = rest[S + 1:S + 1 + n_den]
        rest = rest[S + 1 + n_den:]
        acc_sp, den_sp, zrow, zflat, e_b, m_v = rest[:6]
        set0 = rest[6:11]
        set1 = rest[11:16]
        if tail:
            tbufs = rest[16:22]
            sem0, sem1 = rest[22], rest[23]
        else:
            tbufs = None
            sem0, sem1 = rest[16], rest[17]
        sets = [tuple(set0) + (sem0,), tuple(set1) + (sem1,)]
        c = lax.axis_index("c")
        s = lax.axis_index("s")

        pltpu.sync_copy(m_h, m_v)

        def zfill(i, _):
            zrow[i, pl.ds(0, 16)] = jnp.zeros((16,), jnp.float32)
            zrow[i, pl.ds(16, 16)] = jnp.zeros((16,), jnp.float32)
            return 0
        lax.fori_loop(0, 128, zfill, 0)
        for i in range(8):
            zflat[pl.ds(i * 16, 16)] = jnp.zeros((16,), jnp.float32)

        chunks = [(i * 128, 128) for i in range(24)] + [(3072, 56)]

        def _stripe(off, sz):
            return pl.multiple_of(s * 3128 + off, 8), sz

        def zero_acc():
            for off, sz in chunks:
                o, _ = _stripe(off, sz)
                pltpu.sync_copy(zrow.at[pl.ds(0, sz)],
                                acc_sp.at[pl.ds(o, sz)])

        def zero_den():
            for off, sz in chunks:
                o, _ = _stripe(off, sz)
                pltpu.sync_copy(zflat.at[pl.ds(0, sz)],
                                den_sp.at[pl.ds(o, sz)])

        def copy_acc(out_idx):
            for off, sz in chunks:
                o, _ = _stripe(off, sz)
                pltpu.sync_copy(acc_sp.at[pl.ds(o, sz)],
                                acc_o.at[out_idx, pl.ds(o, sz)])

        def copy_den(den_idx):
            for off, sz in chunks:
                o, _ = _stripe(off, sz)
                pltpu.sync_copy(den_sp.at[pl.ds(o, sz)],
                                den_os[den_idx].at[pl.ds(o, sz)])

        def idx_load(base, bset):
            sb, db = bset[0], bset[1]
            base = pl.multiple_of(base, 8)
            pltpu.sync_copy(src_h.at[pl.ds(base, _KB)], sb)
            pltpu.sync_copy(dst_h.at[pl.ds(base, _KB)], db)

        def g_start(bset, hsl_ref):
            sb, db, ab, bb, rb, sm = bset
            pltpu.async_copy(as_h.at[sb], ab, sm)
            pltpu.async_copy(ad_h.at[db], bb, sm)
            pltpu.async_copy(hsl_ref.at[sb], rb, sm)

        def g_wait(bset, hsl_ref):
            sb, db, ab, bb, rb, sm = bset
            pltpu.make_async_copy(as_h.at[sb], ab, sm).wait()
            pltpu.make_async_copy(ad_h.at[db], bb, sm).wait()
            pltpu.make_async_copy(hsl_ref.at[sb], rb, sm).wait()

        def consume(base, kk, sb, db, ab, bb, eb, rb, with_den):
            base = pl.multiple_of(base, 8)
            m = m_v[pl.ds(0, 16)][0]
            for j in range(kk // 16):
                z = ab[pl.ds(j * 16, 16)] + bb[pl.ds(j * 16, 16)]
                a = jnp.maximum(z, 0.0) + 0.2 * jnp.minimum(z, 0.0)
                e = jnp.exp(a - m)
                gidx = (base + j * 16) + lax.iota(jnp.int32, 16)
                e = jnp.where(gidx < _E_REAL, e, jnp.zeros((16,), jnp.float32))
                eb[pl.ds(j * 16, 16)] = e
            if with_den:
                pltpu.sync_copy(eb, den_sp.at[db], add=True)

            def scale(j, _):
                ev = eb[pl.ds(j * 16, 16)]
                for k2 in range(16):
                    r = j * 16 + k2
                    ek = ev[k2]
                    rb[r, pl.ds(0, 16)] = rb[r, pl.ds(0, 16)] * ek
                    rb[r, pl.ds(16, 16)] = rb[r, pl.ds(16, 16)] * ek
                return 0
            lax.fori_loop(0, kk // 16, scale, 0)
            pltpu.sync_copy(rb, acc_sp.at[db], add=True)

        def run_pass(hsl_ref, out_idx, with_den, den_idx, base0):
            zero_acc()
            if with_den:
                zero_den()
            plsc.subcore_barrier()

            idx_load(base0, sets[0])
            g_start(sets[0], hsl_ref)

            def pair(i, _):
                for bp in range(2):
                    k = 2 * i + bp
                    p, q = sets[bp], sets[1 - bp]
                    idx_load(base0 + (k + 1) * _KB, q)
                    g_start(q, hsl_ref)
                    g_wait(p, hsl_ref)
                    consume(base0 + k * _KB, _KB, p[0], p[1], p[2], p[3],
                            e_b, p[4], with_den)
                return 0
            lax.fori_loop(0, (nblk - 1) // 2, pair, 0)
            p = sets[0]
            g_wait(p, hsl_ref)
            consume(base0 + (nblk - 1) * _KB, _KB, p[0], p[1], p[2], p[3],
                    e_b, p[4], with_den)
            if tail:
                sb, db, ab, bb, eb, rb = tbufs
                base = pl.multiple_of(base0 + nblk * _KB, 8)
                pltpu.sync_copy(src_h.at[pl.ds(base, tail)], sb)
                pltpu.sync_copy(dst_h.at[pl.ds(base, tail)], db)
                pltpu.async_copy(as_h.at[sb], ab, sem0).wait()
                pltpu.async_copy(ad_h.at[db], bb, sem0).wait()
                pltpu.async_copy(hsl_ref.at[sb], rb, sem0).wait()
                consume(base, tail, sb, db, ab, bb, eb, rb, with_den)
            plsc.subcore_barrier()
            copy_acc(out_idx)
            if with_den:
                copy_den(den_idx)
            plsc.subcore_barrier()

        for cc in range(2):
            @pl.when(c == cc)
            def _(cc=cc):
                if split_edges:
                    run_pass(h_refs[0], out_idx=cc, with_den=True, den_idx=cc,
                             base0=(s * 2 + cc) * P)
                else:
                    for si in range(spc):
                        k_idx = cc * spc + si
                        run_pass(h_refs[k_idx], out_idx=k_idx,
                                 with_den=(cc == 0 and si == 0), den_idx=0,
                                 base0=s * P)

    return body


@functools.lru_cache(maxsize=None)
def _make_pool():
    rows_per_w = _NPOOL // 32
    nblk = rows_per_w // _KB
    tail = rows_per_w - nblk * _KB
    mesh = plsc.VectorSubcoreMesh(core_axis_name="c", subcore_axis_name="s")
    out_type = (jax.ShapeDtypeStruct((2, _BP, 128), jnp.float32),
                jax.ShapeDtypeStruct((_BP,), jnp.float32),
                jax.ShapeDtypeStruct((_BP,), jnp.float32))
    scratch = [
        pltpu.VMEM_SHARED((_BP, 128), jnp.float32),
        pltpu.VMEM_SHARED((_BP,), jnp.float32),
        pltpu.VMEM((64, 128), jnp.float32),
        pltpu.VMEM((256,), jnp.float32),
        pltpu.VMEM((_KB, 128), jnp.float32),
        pltpu.VMEM((_KB,), jnp.int32),
        pltpu.VMEM((_KB,), jnp.float32),
        pltpu.VMEM((tail, 128), jnp.float32),
        pltpu.VMEM((tail,), jnp.int32),
        pltpu.VMEM((tail,), jnp.float32),
        pltpu.SemaphoreType.DMA,
    ]

    @functools.partial(pl.kernel, out_type=out_type, mesh=mesh,
                       scratch_types=scratch)
    def body(x_h, b_h, acc_o, cnt_o0, cnt_o1, acc_sp, cnt_sp, zp, zf, rows,
             bidx, ones_v, rows_t, bidx_t, ones_t, sem):
        cnt_os = (cnt_o0, cnt_o1)
        c = lax.axis_index("c")
        s = lax.axis_index("s")

        def zfill(i, _):
            for j in range(8):
                zp[i, pl.ds(j * 16, 16)] = jnp.zeros((16,), jnp.float32)
            return 0
        lax.fori_loop(0, 64, zfill, 0)

        def zfill1(i, _):
            zf[pl.ds(i * 16, 16)] = jnp.zeros((16,), jnp.float32)
            return 0
        lax.fori_loop(0, 16, zfill1, 0)

        for j in range(_KB // 16):
            ones_v[pl.ds(j * 16, 16)] = jnp.ones((16,), jnp.float32)
        for j in range(tail // 16):
            ones_t[pl.ds(j * 16, 16)] = jnp.ones((16,), jnp.float32)

        @pl.when(s == 0)
        def _():
            for i in range(16):
                pltpu.sync_copy(zp, acc_sp.at[pl.ds(i * 64, 64)])
            pltpu.sync_copy(zp.at[pl.ds(0, 8)], acc_sp.at[pl.ds(1024, 8)])
            for i in range(4):
                pltpu.sync_copy(zf, cnt_sp.at[pl.ds(i * 256, 256)])
            pltpu.sync_copy(zf.at[pl.ds(0, 8)], cnt_sp.at[pl.ds(1024, 8)])
        plsc.subcore_barrier()

        def do_block(rb, kk, rbuf, ibuf, obuf):
            rb = pl.multiple_of(rb, 8)
            pltpu.sync_copy(x_h.at[pl.ds(rb, kk)], rbuf)
            pltpu.sync_copy(b_h.at[pl.ds(rb, kk)], ibuf)
            pltpu.sync_copy(rbuf, acc_sp.at[ibuf], add=True)
            pltpu.sync_copy(obuf, cnt_sp.at[ibuf], add=True)

        for cc in range(2):
            @pl.when(c == cc)
            def _(cc=cc):
                base0 = (s * 2 + cc) * rows_per_w

                def blk(i, _):
                    do_block(base0 + i * _KB, _KB, rows, bidx, ones_v)
                    return 0
                lax.fori_loop(0, nblk, blk, 0)
                do_block(base0 + nblk * _KB, tail, rows_t, bidx_t, ones_t)
                plsc.subcore_barrier()

                @pl.when(s == 0)
                def _():
                    pltpu.sync_copy(acc_sp, acc_o.at[cc])
                    pltpu.sync_copy(cnt_sp, cnt_os[cc])

    return body


_BLK_N = 400


def _dot(a, w):
    return jnp.dot(a, w, preferred_element_type=jnp.float32)


def _gat_pre_body(x_ref, w_ref, as_ref, ad_ref, h_ref, aux_ref, mx_ref):
    h = _dot(x_ref[...], w_ref[...])
    h_ref[...] = h
    asum = _dot(h, as_ref[...])
    adsum = _dot(h, ad_ref[...])
    aux = jnp.concatenate([asum, adsum, jnp.zeros_like(asum),
                           jnp.zeros_like(asum)], axis=1)
    aux_ref[...] = aux
    mx_ref[...] = jnp.concatenate([jnp.max(asum).reshape(1, 1, 1),
                                   jnp.max(adsum).reshape(1, 1, 1)], axis=2)


def _gat_pre(x, p):
    n, d_in = x.shape
    dim = p['w'].shape[1]
    grid = (n // _BLK_N,)
    h, aux, mx = pl.pallas_call(
        _gat_pre_body,
        grid=grid,
        in_specs=[pl.BlockSpec((_BLK_N, d_in), lambda i: (i, 0)),
                  pl.BlockSpec((d_in, dim), lambda i: (0, 0)),
                  pl.BlockSpec((dim, 1), lambda i: (0, 0)),
                  pl.BlockSpec((dim, 1), lambda i: (0, 0))],
        out_specs=[pl.BlockSpec((_BLK_N, dim), lambda i: (i, 0)),
                   pl.BlockSpec((_BLK_N, 4), lambda i: (i, 0)),
                   pl.BlockSpec((1, 1, 2), lambda i: (i, 0, 0))],
        out_shape=[jax.ShapeDtypeStruct((n, dim), jnp.float32),
                   jax.ShapeDtypeStruct((n, 4), jnp.float32),
                   jax.ShapeDtypeStruct((grid[0], 1, 2), jnp.float32)],
    )(x, p['w'], p['as'][:, None], p['ad'][:, None])
    return h, aux, mx


def _gat_post_body(acc_ref, den_ref, h_ref, aux_ref, m_ref, b_ref, o_ref):
    m = m_ref[0, 0]
    z = aux_ref[:, 0:1] + aux_ref[:, 1:2]
    e_self = jnp.exp(jnp.where(z > 0, z, 0.2 * z) - m)
    num = acc_ref[...] + e_self * h_ref[...]
    den = den_ref[...] + e_self + 1e-16
    out = num / den + b_ref[...]
    o_ref[...] = jnp.where(out > 0, out, 0.01 * out)


def _gat_post(acc, den, h, aux, m, b):
    n, dim = h.shape
    grid = (n // _BLK_N,)
    return pl.pallas_call(
        _gat_post_body,
        grid=grid,
        in_specs=[pl.BlockSpec((_BLK_N, dim), lambda i: (i, 0)),
                  pl.BlockSpec((_BLK_N, 1), lambda i: (i, 0)),
                  pl.BlockSpec((_BLK_N, dim), lambda i: (i, 0)),
                  pl.BlockSpec((_BLK_N, 4), lambda i: (i, 0)),
                  pl.BlockSpec((1, 1), lambda i: (0, 0)),
                  pl.BlockSpec((1, dim), lambda i: (0, 0))],
        out_specs=pl.BlockSpec((_BLK_N, dim), lambda i: (i, 0)),
        out_shape=jax.ShapeDtypeStruct((n, dim), jnp.float32),
    )(acc, den[:, None], h, aux, m[None, None], b[None, :])


def _fem_head_body(ps_ref, cnt_ref, w1, b1, g1, bb1, w2, b2, o_ref):
    pooled = ps_ref[...] / jnp.maximum(cnt_ref[...], 1.0)
    h = _dot(pooled, w1[...]) + b1[...]
    h = h * _BN_SCALE * g1[...] + bb1[...]
    h = jnp.where(h > 0, h, 0.01 * h)
    o_ref[...] = _dot(h, w2[...]) + b2[...]


def _fem_head(psum, cnt, p):
    grid = (_B // _BLK_B,)
    vec = lambda a: a[None, :]
    return pl.pallas_call(
        _fem_head_body,
        grid=grid,
        in_specs=[pl.BlockSpec((_BLK_B, 128), lambda i: (i, 0)),
                  pl.BlockSpec((_BLK_B, 1), lambda i: (i, 0))] +
                 [pl.BlockSpec(s, lambda i: (0, 0)) for s in
                  [(128, 128), (1, 128), (1, 128), (1, 128), (128, 128), (1, 128)]],
        out_specs=pl.BlockSpec((_BLK_B, 128), lambda i: (i, 0)),
        out_shape=jax.ShapeDtypeStruct((_B, 128), jnp.float32),
    )(psum, cnt[:, None], p['fc1']['w'], vec(p['fc1']['b']),
      vec(p['fc1_bn']['g']), vec(p['fc1_bn']['b']),
      p['fc2']['w'], vec(p['fc2']['b']))


def _fp_body(x_ref, w1, b1, g1, bb1, w2, b2, g2, bb2, o_ref):
    h = _dot(x_ref[...], w1[...]) + b1[...]
    h = jnp.maximum(h * _BN_SCALE * g1[...] + bb1[...], 0.0)
    h = _dot(h, w2[...]) + b2[...]
    o_ref[...] = jnp.maximum(h * _BN_SCALE * g2[...] + bb2[...], 0.0)


def _fp_enc_tc(x, p):
    grid = (_B // _BLK_B,)
    vec = lambda a: a[None, :]
    return pl.pallas_call(
        _fp_body,
        grid=grid,
        in_specs=[pl.BlockSpec((_BLK_B, 2048), lambda i: (i, 0))] +
                 [pl.BlockSpec(s, lambda i: (0, 0)) for s in
                  [(2048, 1024), (1, 1024), (1, 1024), (1, 1024),
                   (1024, 128), (1, 128), (1, 128), (1, 128)]],
        out_specs=pl.BlockSpec((_BLK_B, 128), lambda i: (i, 0)),
        out_shape=jax.ShapeDtypeStruct((_B, 128), jnp.float32),
    )(x, p['l1']['w'], vec(p['l1']['b']), vec(p['bn1']['g']), vec(p['bn1']['b']),
      p['l2']['w'], vec(p['l2']['b']), vec(p['bn2']['g']), vec(p['bn2']['b']))


def _cell_body(x_ref, w1, b1, g1, bb1, w2, b2, g2, bb2, w3, b3, o_ref):
    x = x_ref[...]
    nrm = jnp.sqrt(jnp.sum(x * x, axis=1, keepdims=True))
    x = x / jnp.maximum(nrm, 1e-12)
    h = _dot(x, w1[...]) + b1[...]
    h = h * _BN_SCALE * g1[...] + bb1[...]
    h = jnp.where(h > 0, h, 0.01 * h)
    h = _dot(h, w2[...]) + b2[...]
    h = h * _BN_SCALE * g2[...] + bb2[...]
    h = jnp.where(h > 0, h, 0.01 * h)
    o_ref[...] = _dot(h, w3[...]) + b3[...]


def _cell_enc_tc(x, p):
    grid = (_B // _BLK_B,)
    vec = lambda a: a[None, :]
    return pl.pallas_call(
        _cell_body,
        grid=grid,
        in_specs=[pl.BlockSpec((_BLK_B, 954), lambda i: (i, 0))] +
                 [pl.BlockSpec(s, lambda i: (0, 0)) for s in
                  [(954, 256), (1, 256), (1, 256), (1, 256),
                   (256, 128), (1, 128), (1, 128), (1, 128),
                   (128, 128), (1, 128)]],
        out_specs=pl.BlockSpec((_BLK_B, 128), lambda i: (i, 0)),
        out_shape=jax.ShapeDtypeStruct((_B, 128), jnp.float32),
    )(x, p['l1']['w'], vec(p['l1']['b']), vec(p['bn1']['g']), vec(p['bn1']['b']),
      p['l2']['w'], vec(p['l2']['b']), vec(p['bn2']['g']), vec(p['bn2']['b']),
      p['l3']['w'], vec(p['l3']['b']))


def _gat_layer_sc(x, src_p, dst_p, p):
    h, aux, mx = _gat_pre(x, p)
    dim = h.shape[1]
    S = dim // 32
    asum = aux[:, 0]
    adsum = aux[:, 1]
    mb = jnp.max(mx[:, 0, 0]) + jnp.max(mx[:, 0, 1])
    m = jnp.where(mb > 0, mb, 0.2 * mb)
    marr = jnp.full((16,), m, jnp.float32)
    hsl = [h[:, 32 * i:32 * (i + 1)] for i in range(S)]
    res = _make_edge_aggr(dim)(src_p, dst_p, asum, adsum, marr, *hsl)
    acc_p, den_p = res[0], res[1:]
    if S == 1:
        acc = acc_p[0] + acc_p[1]
        den = den_p[0] + den_p[1]
    else:
        acc = jnp.concatenate([acc_p[i] for i in range(S)], axis=1)
        den = den_p[0]
    return _gat_post(acc[:_NN], den[:_NN], h, aux, m, p['b'])


def _fem(x, ei, batch, p, b):
    src_p = jnp.pad(ei[0], (0, _E_PAD - _E_REAL))
    dst_p = jnp.pad(ei[1], (0, _E_PAD - _E_REAL))
    for gp in p['gat']:
        x = _gat_layer_sc(x, src_p, dst_p, gp)
    xp = jnp.pad(x, ((0, _NPOOL - _NN), (0, 0)))
    bp = jnp.pad(batch, (0, _NPOOL - _NN), constant_values=_B)
    acc_p, cnt0, cnt1 = _make_pool()(xp, bp)
    pooled_sum = (acc_p[0] + acc_p[1])[:_B]
    cnt = (cnt0 + cnt1)[:_B]
    return _fem_head(pooled_sum, cnt, p)


def kernel(x1, edge_index1, batch1, fp1, x2, edge_index2, batch2, fp2, cell, params):
    b = fp1.shape[0]
    x1g = _fem(x1, edge_index1, batch1, params['fem1'], b)
    x2g = _fem(x2, edge_index2, batch2, params['fem2'], b)
    f1 = _fp_enc_tc(fp1, params['fp'])
    f2 = _fp_enc_tc(fp2, params['fp'])
    cv = _cell_enc_tc(cell, params['cell'])
    return _tail(x1g, x2g, f1, f2, cv, params)
